# Initial kernel scaffold; baseline (speedup 1.0000x reference)
import jax
import jax.numpy as jnp
from jax.experimental import pallas as pl


def kernel(x, edge_index, W1, b1, W2, b2, W3, b3, W4, b4):
    raise NotImplementedError("write your pallas kernel here")


# speedup vs baseline: 16.0200x; 16.0200x over previous
import functools

import jax
import jax.numpy as jnp
from jax import lax
from jax.experimental import pallas as pl
from jax.experimental.pallas import tpu as pltpu
from jax.experimental.pallas import tpu_sc as plsc

N = 100000
E = 3200000
NC = 2
NS = 16
LANES = 128
GROUP = 8

NPAD = 100096
EPW_PAD = 102400
EPAD = EPW_PAD * NC * NS
EROWS = EPAD // LANES
ROWS_PER_TILE = EROWS // (NC * NS)
STEPS = ROWS_PER_TILE // GROUP
INIT_ROWS = NPAD // NS

_mesh = plsc.VectorSubcoreMesh(
    core_axis_name="c", subcore_axis_name="s", num_cores=NC, num_subcores=NS)


def _edge_pass_16(src2d, dst2d, y, zeros2d, gather=True):
  @functools.partial(
      pl.kernel,
      out_type=[jax.ShapeDtypeStruct((NPAD, 16), jnp.float32),
                jax.ShapeDtypeStruct((NPAD, 16), jnp.float32)],
      mesh=_mesh,
      compiler_params=pltpu.CompilerParams(use_tc_tiling_on_sc=False),
      scratch_types=[
          pltpu.VMEM_SHARED((NPAD, 16), jnp.float32),
          pltpu.VMEM((GROUP, LANES), jnp.int32),
          pltpu.VMEM((GROUP, LANES), jnp.int32),
          pltpu.VMEM((GROUP * LANES, 16), jnp.float32),
      ],
  )
  def k(src_hbm, dst_hbm, y_hbm, zero_hbm, za_hbm, zb_hbm, acc, sv, dv, rows):
    c = lax.axis_index("c")
    s = lax.axis_index("s")
    ib = s * INIT_ROWS

    if gather:
      @pl.when(c == 0)
      def _():
        pltpu.sync_copy(y_hbm.at[pl.ds(ib, INIT_ROWS)], acc.at[pl.ds(ib, INIT_ROWS)])

      @pl.when(c != 0)
      def _():
        pltpu.sync_copy(zero_hbm.at[pl.ds(ib, INIT_ROWS)], acc.at[pl.ds(ib, INIT_ROWS)])
    else:
      pltpu.sync_copy(zero_hbm.at[pl.ds(ib, INIT_ROWS)], acc.at[pl.ds(ib, INIT_ROWS)])
      pltpu.sync_copy(y_hbm, rows)

    plsc.subcore_barrier()

    base = (c * NS + s) * ROWS_PER_TILE

    def step(g, _):
      r0 = base + g * GROUP
      pltpu.sync_copy(dst_hbm.at[pl.ds(r0, GROUP)], dv)
      if gather:
        pltpu.sync_copy(src_hbm.at[pl.ds(r0, GROUP)], sv)
        for j in range(GROUP):
          pltpu.sync_copy(y_hbm.at[sv.at[j]], rows.at[pl.ds(j * LANES, LANES)])
      for j in range(GROUP):
        pltpu.sync_copy(rows.at[pl.ds(j * LANES, LANES)], acc.at[dv.at[j]],
                        add=True)
      return _

    lax.fori_loop(0, STEPS, step, None)
    plsc.subcore_barrier()

    @pl.when(c == 0)
    def _():
      pltpu.sync_copy(acc.at[pl.ds(ib, INIT_ROWS)], za_hbm.at[pl.ds(ib, INIT_ROWS)])

    @pl.when(c != 0)
    def _():
      pltpu.sync_copy(acc.at[pl.ds(ib, INIT_ROWS)], zb_hbm.at[pl.ds(ib, INIT_ROWS)])

  return k(src2d, dst2d, y, zeros2d)


_BLK = 128
_GRID = NPAD // _BLK


def _t0_body(d0, d1, x, w, dinv_o, y_o):
  deg = d0[...][:, :1] + d1[...][:, :1] + 1.0
  dinv = lax.rsqrt(deg)
  dinv_o[...] = dinv
  y_o[...] = dinv * jnp.dot(x[...], w[...], preferred_element_type=jnp.float32)


def _t0(deg0, deg1, x, w1):
  return pl.pallas_call(
      _t0_body,
      grid=(_GRID,),
      in_specs=[
          pl.BlockSpec((_BLK, 16), lambda i: (i, 0)),
          pl.BlockSpec((_BLK, 16), lambda i: (i, 0)),
          pl.BlockSpec((_BLK, 3), lambda i: (i, 0)),
          pl.BlockSpec((3, 16), lambda i: (0, 0)),
      ],
      out_specs=[
          pl.BlockSpec((_BLK, 1), lambda i: (i, 0)),
          pl.BlockSpec((_BLK, 16), lambda i: (i, 0)),
      ],
      out_shape=[jax.ShapeDtypeStruct((NPAD, 1), jnp.float32),
                 jax.ShapeDtypeStruct((NPAD, 16), jnp.float32)],
  )(deg0, deg1, x, w1)


def _tmid_body(za, zb, dinv, b, w, y_o):
  h = jnp.maximum(dinv[...] * (za[...] + zb[...]) + b[...], 0.0)
  y_o[...] = dinv[...] * jnp.dot(h, w[...], preferred_element_type=jnp.float32)


def _tmid(za, zb, dinv, b, w):
  return pl.pallas_call(
      _tmid_body,
      grid=(_GRID,),
      in_specs=[
          pl.BlockSpec((_BLK, 16), lambda i: (i, 0)),
          pl.BlockSpec((_BLK, 16), lambda i: (i, 0)),
          pl.BlockSpec((_BLK, 1), lambda i: (i, 0)),
          pl.BlockSpec((1, 16), lambda i: (0, 0)),
          pl.BlockSpec((16, 16), lambda i: (0, 0)),
      ],
      out_specs=pl.BlockSpec((_BLK, 16), lambda i: (i, 0)),
      out_shape=jax.ShapeDtypeStruct((NPAD, 16), jnp.float32),
  )(za, zb, dinv, b, w)


def _t4_body(za, zb, dinv, b, o):
  h = dinv[...] * (za[...][:, :1] + zb[...][:, :1]) + b[...]
  m = jnp.max(h, axis=1, keepdims=True)
  o[...] = (h - m) - jnp.log(jnp.sum(jnp.exp(h - m), axis=1, keepdims=True))


def _t4(za, zb, dinv, b4):
  return pl.pallas_call(
      _t4_body,
      grid=(_GRID,),
      in_specs=[
          pl.BlockSpec((_BLK, 16), lambda i: (i, 0)),
          pl.BlockSpec((_BLK, 16), lambda i: (i, 0)),
          pl.BlockSpec((_BLK, 1), lambda i: (i, 0)),
          pl.BlockSpec((1, 1), lambda i: (0, 0)),
      ],
      out_specs=pl.BlockSpec((_BLK, 1), lambda i: (i, 0)),
      out_shape=jax.ShapeDtypeStruct((NPAD, 1), jnp.float32),
  )(za, zb, dinv, b4)


def kernel(x, edge_index, W1, b1, W2, b2, W3, b3, W4, b4):
  src = edge_index[0].astype(jnp.int32)
  dst = edge_index[1].astype(jnp.int32)
  src2d = jnp.reshape(jnp.pad(src, (0, EPAD - E)), (EROWS, LANES))
  dst2d = jnp.reshape(jnp.pad(dst, (0, EPAD - E), constant_values=N),
                      (EROWS, LANES))
  x_pad = jnp.pad(x, ((0, NPAD - N), (0, 0)))
  zeros2d = jnp.zeros((NPAD, 16), jnp.float32)
  ones16 = jnp.ones((GROUP * LANES, 16), jnp.float32)

  deg0, deg1 = _edge_pass_16(src2d, dst2d, ones16, zeros2d, gather=False)
  dinv, y1 = _t0(deg0, deg1, x_pad, W1)

  za, zb = _edge_pass_16(src2d, dst2d, y1, zeros2d)
  y2 = _tmid(za, zb, dinv, b1.reshape(1, 16), W2)
  za, zb = _edge_pass_16(src2d, dst2d, y2, zeros2d)
  y3 = _tmid(za, zb, dinv, b2.reshape(1, 16), W3)
  za, zb = _edge_pass_16(src2d, dst2d, y3, zeros2d)
  w4p = jnp.pad(W4, ((0, 0), (0, 15)))
  y4 = _tmid(za, zb, dinv, b3.reshape(1, 16), w4p)

  z4a, z4b = _edge_pass_16(src2d, dst2d, y4, zeros2d)
  out = _t4(z4a, z4b, dinv, b4.reshape(1, 1))
  return out[:N]


# speedup vs baseline: 21.4322x; 1.3378x over previous
import functools

import jax
import jax.numpy as jnp
from jax import lax
from jax.experimental import pallas as pl
from jax.experimental.pallas import tpu as pltpu
from jax.experimental.pallas import tpu_sc as plsc

N = 100000
E = 3200000
NC = 2
NS = 16
LANES = 128
GROUP = 8

NPAD = 100096
EPW_PAD = 102400
EPAD = EPW_PAD * NC * NS
EROWS = EPAD // LANES
ROWS_PER_TILE = EROWS // (NC * NS)
STEPS = ROWS_PER_TILE // GROUP
INIT_ROWS = NPAD // NS

_mesh = plsc.VectorSubcoreMesh(
    core_axis_name="c", subcore_axis_name="s", num_cores=NC, num_subcores=NS)


def _edge_pass_16(src2d, dst2d, y, zeros2d, gather=True):
  @functools.partial(
      pl.kernel,
      out_type=[jax.ShapeDtypeStruct((NPAD, 16), jnp.float32),
                jax.ShapeDtypeStruct((NPAD, 16), jnp.float32)],
      mesh=_mesh,
      compiler_params=pltpu.CompilerParams(use_tc_tiling_on_sc=False),
      scratch_types=[
          pltpu.VMEM_SHARED((NPAD, 16), jnp.float32),
          pltpu.VMEM((GROUP, LANES), jnp.int32),
          pltpu.VMEM((GROUP, LANES), jnp.int32),
          pltpu.VMEM((GROUP * LANES, 16), jnp.float32),
      ],
  )
  def k(src_hbm, dst_hbm, y_hbm, zero_hbm, za_hbm, zb_hbm, acc, sv, dv, rows):
    c = lax.axis_index("c")
    s = lax.axis_index("s")
    ib = s * INIT_ROWS

    if gather:
      @pl.when(c == 0)
      def _():
        pltpu.sync_copy(y_hbm.at[pl.ds(ib, INIT_ROWS)], acc.at[pl.ds(ib, INIT_ROWS)])

      @pl.when(c != 0)
      def _():
        pltpu.sync_copy(zero_hbm.at[pl.ds(ib, INIT_ROWS)], acc.at[pl.ds(ib, INIT_ROWS)])
    else:
      pltpu.sync_copy(zero_hbm.at[pl.ds(ib, INIT_ROWS)], acc.at[pl.ds(ib, INIT_ROWS)])
      pltpu.sync_copy(y_hbm, rows)

    plsc.subcore_barrier()

    base = (c * NS + s) * ROWS_PER_TILE

    def step(g, _):
      r0 = base + g * GROUP
      pltpu.sync_copy(dst_hbm.at[pl.ds(r0, GROUP)], dv)
      if gather:
        pltpu.sync_copy(src_hbm.at[pl.ds(r0, GROUP)], sv)
        for j in range(GROUP):
          pltpu.sync_copy(y_hbm.at[sv.at[j]], rows.at[pl.ds(j * LANES, LANES)])
      for j in range(GROUP):
        pltpu.sync_copy(rows.at[pl.ds(j * LANES, LANES)], acc.at[dv.at[j]],
                        add=True)
      return _

    lax.fori_loop(0, STEPS, step, None)
    plsc.subcore_barrier()

    @pl.when(c == 0)
    def _():
      pltpu.sync_copy(acc.at[pl.ds(ib, INIT_ROWS)], za_hbm.at[pl.ds(ib, INIT_ROWS)])

    @pl.when(c != 0)
    def _():
      pltpu.sync_copy(acc.at[pl.ds(ib, INIT_ROWS)], zb_hbm.at[pl.ds(ib, INIT_ROWS)])

  return k(src2d, dst2d, y, zeros2d)


_BLK = 6256
_GRID = NPAD // _BLK


def _t0_body(d0, d1, x, w, dinv_o, y_o):
  deg = d0[...][:, :1] + d1[...][:, :1] + 1.0
  dinv = lax.rsqrt(deg)
  dinv_o[...] = dinv
  y_o[...] = dinv * jnp.dot(x[...], w[...], preferred_element_type=jnp.float32)


def _t0(deg0, deg1, x, w1):
  return pl.pallas_call(
      _t0_body,
      grid=(_GRID,),
      in_specs=[
          pl.BlockSpec((_BLK, 16), lambda i: (i, 0)),
          pl.BlockSpec((_BLK, 16), lambda i: (i, 0)),
          pl.BlockSpec((_BLK, 3), lambda i: (i, 0)),
          pl.BlockSpec((3, 16), lambda i: (0, 0)),
      ],
      out_specs=[
          pl.BlockSpec((_BLK, 1), lambda i: (i, 0)),
          pl.BlockSpec((_BLK, 16), lambda i: (i, 0)),
      ],
      out_shape=[jax.ShapeDtypeStruct((NPAD, 1), jnp.float32),
                 jax.ShapeDtypeStruct((NPAD, 16), jnp.float32)],
  )(deg0, deg1, x, w1)


def _tmid_body(za, zb, dinv, b, w, y_o):
  h = jnp.maximum(dinv[...] * (za[...] + zb[...]) + b[...], 0.0)
  y_o[...] = dinv[...] * jnp.dot(h, w[...], preferred_element_type=jnp.float32)


def _tmid(za, zb, dinv, b, w):
  return pl.pallas_call(
      _tmid_body,
      grid=(_GRID,),
      in_specs=[
          pl.BlockSpec((_BLK, 16), lambda i: (i, 0)),
          pl.BlockSpec((_BLK, 16), lambda i: (i, 0)),
          pl.BlockSpec((_BLK, 1), lambda i: (i, 0)),
          pl.BlockSpec((1, 16), lambda i: (0, 0)),
          pl.BlockSpec((16, 16), lambda i: (0, 0)),
      ],
      out_specs=pl.BlockSpec((_BLK, 16), lambda i: (i, 0)),
      out_shape=jax.ShapeDtypeStruct((NPAD, 16), jnp.float32),
  )(za, zb, dinv, b, w)


def _t4_body(za, zb, dinv, b, o):
  h = dinv[...] * (za[...][:, :1] + zb[...][:, :1]) + b[...]
  m = jnp.max(h, axis=1, keepdims=True)
  o[...] = (h - m) - jnp.log(jnp.sum(jnp.exp(h - m), axis=1, keepdims=True))


def _t4(za, zb, dinv, b4):
  return pl.pallas_call(
      _t4_body,
      grid=(_GRID,),
      in_specs=[
          pl.BlockSpec((_BLK, 16), lambda i: (i, 0)),
          pl.BlockSpec((_BLK, 16), lambda i: (i, 0)),
          pl.BlockSpec((_BLK, 1), lambda i: (i, 0)),
          pl.BlockSpec((1, 1), lambda i: (0, 0)),
      ],
      out_specs=pl.BlockSpec((_BLK, 1), lambda i: (i, 0)),
      out_shape=jax.ShapeDtypeStruct((NPAD, 1), jnp.float32),
  )(za, zb, dinv, b4)


def kernel(x, edge_index, W1, b1, W2, b2, W3, b3, W4, b4):
  src = edge_index[0].astype(jnp.int32)
  dst = edge_index[1].astype(jnp.int32)
  src2d = jnp.reshape(jnp.pad(src, (0, EPAD - E)), (EROWS, LANES))
  dst2d = jnp.reshape(jnp.pad(dst, (0, EPAD - E), constant_values=N),
                      (EROWS, LANES))
  x_pad = jnp.pad(x, ((0, NPAD - N), (0, 0)))
  zeros2d = jnp.zeros((NPAD, 16), jnp.float32)
  ones16 = jnp.ones((GROUP * LANES, 16), jnp.float32)

  deg0, deg1 = _edge_pass_16(src2d, dst2d, ones16, zeros2d, gather=False)
  dinv, y1 = _t0(deg0, deg1, x_pad, W1)

  za, zb = _edge_pass_16(src2d, dst2d, y1, zeros2d)
  y2 = _tmid(za, zb, dinv, b1.reshape(1, 16), W2)
  za, zb = _edge_pass_16(src2d, dst2d, y2, zeros2d)
  y3 = _tmid(za, zb, dinv, b2.reshape(1, 16), W3)
  za, zb = _edge_pass_16(src2d, dst2d, y3, zeros2d)
  w4p = jnp.pad(W4, ((0, 0), (0, 15)))
  y4 = _tmid(za, zb, dinv, b3.reshape(1, 16), w4p)

  z4a, z4b = _edge_pass_16(src2d, dst2d, y4, zeros2d)
  out = _t4(z4a, z4b, dinv, b4.reshape(1, 1))
  return out[:N]


# speedup vs baseline: 29.4907x; 1.3760x over previous
import functools

import jax
import jax.numpy as jnp
from jax import lax
from jax.experimental import pallas as pl
from jax.experimental.pallas import tpu as pltpu
from jax.experimental.pallas import tpu_sc as plsc

N = 100000
E = 3200000
NC = 2
NS = 16
LANES = 128
GROUP = 4
Q = 3

NPAD = 100096
ROWS_PER_TILE = 804
EPW_PAD = ROWS_PER_TILE * LANES
EPAD = EPW_PAD * NC * NS
EROWS = EPAD // LANES
STEPS = ROWS_PER_TILE // GROUP
SUPER = ROWS_PER_TILE // (Q * GROUP)
INIT_ROWS = NPAD // NS

_mesh = plsc.VectorSubcoreMesh(
    core_axis_name="c", subcore_axis_name="s", num_cores=NC, num_subcores=NS)


def _edge_pass_16(src2d, dst2d, y, zeros2d, gather=True):
  @functools.partial(
      pl.kernel,
      out_type=[jax.ShapeDtypeStruct((NPAD, 16), jnp.float32),
                jax.ShapeDtypeStruct((NPAD, 16), jnp.float32)],
      mesh=_mesh,
      compiler_params=pltpu.CompilerParams(use_tc_tiling_on_sc=False),
      scratch_types=[
          pltpu.VMEM_SHARED((NPAD, 16), jnp.float32),
          pltpu.VMEM((Q * GROUP, LANES), jnp.int32),
          pltpu.VMEM((Q * GROUP, LANES), jnp.int32),
          pltpu.VMEM((Q * GROUP * LANES, 16), jnp.float32),
          [pltpu.SemaphoreType.DMA] * Q,
          [pltpu.SemaphoreType.DMA] * Q,
      ],
  )
  def k(src_hbm, dst_hbm, y_hbm, zero_hbm, za_hbm, zb_hbm, acc, sv, dv, rows,
        gsems, ssems):
    c = lax.axis_index("c")
    s = lax.axis_index("s")
    ib = s * INIT_ROWS

    if gather:
      @pl.when(c == 0)
      def _():
        pltpu.sync_copy(y_hbm.at[pl.ds(ib, INIT_ROWS)], acc.at[pl.ds(ib, INIT_ROWS)])

      @pl.when(c != 0)
      def _():
        pltpu.sync_copy(zero_hbm.at[pl.ds(ib, INIT_ROWS)], acc.at[pl.ds(ib, INIT_ROWS)])
    else:
      pltpu.sync_copy(zero_hbm.at[pl.ds(ib, INIT_ROWS)], acc.at[pl.ds(ib, INIT_ROWS)])
      pltpu.sync_copy(y_hbm, rows.at[pl.ds(0, GROUP * LANES)])

    plsc.subcore_barrier()

    base = (c * NS + s) * ROWS_PER_TILE

    if gather:
      def step(g, _):
        gds = []
        for q in range(Q):
          r0 = base + (g * Q + q) * GROUP
          qg = q * GROUP
          pltpu.sync_copy(src_hbm.at[pl.ds(r0, GROUP)],
                          sv.at[pl.ds(qg, GROUP)])
          pltpu.sync_copy(dst_hbm.at[pl.ds(r0, GROUP)],
                          dv.at[pl.ds(qg, GROUP)])
          for j in range(GROUP):
            gds.append(pltpu.async_copy(
                y_hbm.at[sv.at[qg + j]],
                rows.at[pl.ds((qg + j) * LANES, LANES)], gsems[q]))
        sds = []
        for q in range(Q):
          qg = q * GROUP
          for j in range(GROUP):
            gds[qg + j].wait()
          for j in range(GROUP):
            sds.append(pltpu.async_copy(
                rows.at[pl.ds((qg + j) * LANES, LANES)],
                acc.at[dv.at[qg + j]], ssems[q], add=True))
        for d in sds:
          d.wait()
        return _

      lax.fori_loop(0, SUPER, step, None)
    else:
      def step(g, _):
        r0 = base + g * GROUP
        pltpu.sync_copy(dst_hbm.at[pl.ds(r0, GROUP)], dv.at[pl.ds(0, GROUP)])
        for j in range(GROUP):
          pltpu.sync_copy(rows.at[pl.ds(j * LANES, LANES)], acc.at[dv.at[j]],
                          add=True)
        return _

      lax.fori_loop(0, STEPS, step, None)
    plsc.subcore_barrier()

    @pl.when(c == 0)
    def _():
      pltpu.sync_copy(acc.at[pl.ds(ib, INIT_ROWS)], za_hbm.at[pl.ds(ib, INIT_ROWS)])

    @pl.when(c != 0)
    def _():
      pltpu.sync_copy(acc.at[pl.ds(ib, INIT_ROWS)], zb_hbm.at[pl.ds(ib, INIT_ROWS)])

  return k(src2d, dst2d, y, zeros2d)


_BLK = 6256
_GRID = NPAD // _BLK


def _t0_body(d0, d1, x, w, dinv_o, y_o):
  deg = d0[...][:, :1] + d1[...][:, :1] + 1.0
  dinv = lax.rsqrt(deg)
  dinv_o[...] = dinv
  y_o[...] = dinv * jnp.dot(x[...], w[...], preferred_element_type=jnp.float32)


def _t0(deg0, deg1, x, w1):
  return pl.pallas_call(
      _t0_body,
      grid=(_GRID,),
      in_specs=[
          pl.BlockSpec((_BLK, 16), lambda i: (i, 0)),
          pl.BlockSpec((_BLK, 16), lambda i: (i, 0)),
          pl.BlockSpec((_BLK, 3), lambda i: (i, 0)),
          pl.BlockSpec((3, 16), lambda i: (0, 0)),
      ],
      out_specs=[
          pl.BlockSpec((_BLK, 1), lambda i: (i, 0)),
          pl.BlockSpec((_BLK, 16), lambda i: (i, 0)),
      ],
      out_shape=[jax.ShapeDtypeStruct((NPAD, 1), jnp.float32),
                 jax.ShapeDtypeStruct((NPAD, 16), jnp.float32)],
  )(deg0, deg1, x, w1)


def _tmid_body(za, zb, dinv, b, w, y_o):
  h = jnp.maximum(dinv[...] * (za[...] + zb[...]) + b[...], 0.0)
  y_o[...] = dinv[...] * jnp.dot(h, w[...], preferred_element_type=jnp.float32)


def _tmid(za, zb, dinv, b, w):
  return pl.pallas_call(
      _tmid_body,
      grid=(_GRID,),
      in_specs=[
          pl.BlockSpec((_BLK, 16), lambda i: (i, 0)),
          pl.BlockSpec((_BLK, 16), lambda i: (i, 0)),
          pl.BlockSpec((_BLK, 1), lambda i: (i, 0)),
          pl.BlockSpec((1, 16), lambda i: (0, 0)),
          pl.BlockSpec((16, 16), lambda i: (0, 0)),
      ],
      out_specs=pl.BlockSpec((_BLK, 16), lambda i: (i, 0)),
      out_shape=jax.ShapeDtypeStruct((NPAD, 16), jnp.float32),
  )(za, zb, dinv, b, w)


def _t4_body(za, zb, dinv, b, o):
  h = dinv[...] * (za[...][:, :1] + zb[...][:, :1]) + b[...]
  m = jnp.max(h, axis=1, keepdims=True)
  o[...] = (h - m) - jnp.log(jnp.sum(jnp.exp(h - m), axis=1, keepdims=True))


def _t4(za, zb, dinv, b4):
  return pl.pallas_call(
      _t4_body,
      grid=(_GRID,),
      in_specs=[
          pl.BlockSpec((_BLK, 16), lambda i: (i, 0)),
          pl.BlockSpec((_BLK, 16), lambda i: (i, 0)),
          pl.BlockSpec((_BLK, 1), lambda i: (i, 0)),
          pl.BlockSpec((1, 1), lambda i: (0, 0)),
      ],
      out_specs=pl.BlockSpec((_BLK, 1), lambda i: (i, 0)),
      out_shape=jax.ShapeDtypeStruct((NPAD, 1), jnp.float32),
  )(za, zb, dinv, b4)


def kernel(x, edge_index, W1, b1, W2, b2, W3, b3, W4, b4):
  src = edge_index[0].astype(jnp.int32)
  dst = edge_index[1].astype(jnp.int32)
  src2d = jnp.reshape(jnp.pad(src, (0, EPAD - E)), (EROWS, LANES))
  dst2d = jnp.reshape(jnp.pad(dst, (0, EPAD - E), constant_values=N),
                      (EROWS, LANES))
  x_pad = jnp.pad(x, ((0, NPAD - N), (0, 0)))
  zeros2d = jnp.zeros((NPAD, 16), jnp.float32)
  ones16 = jnp.ones((GROUP * LANES, 16), jnp.float32)

  deg0, deg1 = _edge_pass_16(src2d, dst2d, ones16, zeros2d, gather=False)
  dinv, y1 = _t0(deg0, deg1, x_pad, W1)

  za, zb = _edge_pass_16(src2d, dst2d, y1, zeros2d)
  y2 = _tmid(za, zb, dinv, b1.reshape(1, 16), W2)
  za, zb = _edge_pass_16(src2d, dst2d, y2, zeros2d)
  y3 = _tmid(za, zb, dinv, b2.reshape(1, 16), W3)
  za, zb = _edge_pass_16(src2d, dst2d, y3, zeros2d)
  w4p = jnp.pad(W4, ((0, 0), (0, 15)))
  y4 = _tmid(za, zb, dinv, b3.reshape(1, 16), w4p)

  z4a, z4b = _edge_pass_16(src2d, dst2d, y4, zeros2d)
  out = _t4(z4a, z4b, dinv, b4.reshape(1, 1))
  return out[:N]
